# baseline (speedup 1.0000x reference)
import jax
import jax.numpy as jnp
from jax import lax
from jax.experimental import pallas as pl
from jax.experimental.pallas import tpu as pltpu
from jax.experimental.pallas import tpu_sc as plsc

B = 16384
D = 128
R = 500
BLK = 128
NPAD = B + BLK

NC = 2
NS = 16
LANES = 16
NWK = NC * NS
BPW = B // NWK
CH = 128
NCH = BPW // CH

def _sc_mesh():
    return plsc.VectorSubcoreMesh(
        core_axis_name="c", subcore_axis_name="s", num_cores=NC, num_subcores=NS
    )


def _sc_gather_body(table, hidx, tidx, hout, tout, idx_v, rows_v, sem):
    wid = lax.axis_index("s") * NC + lax.axis_index("c")
    base = wid * BPW
    for ids_hbm, out_hbm in ((hidx, hout), (tidx, tout)):
        pltpu.sync_copy(ids_hbm.at[wid], idx_v)
        cps = [
            pltpu.async_copy(
                table.at[idx_v.at[k]], rows_v.at[pl.ds(k * CH, CH)], sem
            )
            for k in range(NCH)
        ]
        for cp in cps:
            cp.wait()
        pltpu.sync_copy(rows_v, out_hbm.at[pl.ds(base, BPW)])


def _sc_unperm_body(scores, invp, out, sc_v, idx_v, out_v):
    wid = lax.axis_index("s") * NC + lax.axis_index("c")
    base = wid * BPW
    pltpu.sync_copy(scores, sc_v)
    pltpu.sync_copy(invp.at[pl.ds(base, BPW)], idx_v)
    for j in range(BPW // LANES):
        iv = idx_v[pl.ds(j * LANES, LANES)]
        out_v[pl.ds(j * LANES, LANES)] = plsc.load_gather(sc_v, [iv])
    pltpu.sync_copy(out_v, out.at[pl.ds(base, BPW)])


WORK_N = B // BLK + R


def _tc_body(rel_ref, win_ref, lo_ref, hi_ref, m_ref, h_ref, t_ref, out_ref):
    j = pl.program_id(0)
    w = win_ref[j]
    first = jnp.logical_or(j == 0, w != win_ref[jnp.maximum(j - 1, 0)])
    lo = lo_ref[j]
    hi = hi_ref[j]
    hs = h_ref[...]
    ts = t_ref[...]
    m = m_ref[0]
    proj = jax.lax.dot_general(
        hs, m, (((1,), (0,)), ((), ())),
        preferred_element_type=jnp.float32,
    )
    s = jnp.sum(proj * ts, axis=1)
    idx = w * BLK + jax.lax.broadcasted_iota(jnp.int32, (BLK,), 0)
    contrib = jnp.where((idx >= lo) & (idx < hi), s, 0.0)
    out_ref[...] = jnp.where(first, contrib, out_ref[...] + contrib)


def _grouped_scores(item_rel, item_win, item_lo, item_hi, m3, h_pad, t_pad):
    grid_spec = pltpu.PrefetchScalarGridSpec(
        num_scalar_prefetch=4,
        grid=(WORK_N,),
        in_specs=[
            pl.BlockSpec((1, D, D), lambda j, rel, win, lo, hi: (rel[j], 0, 0)),
            pl.BlockSpec((BLK, D), lambda j, rel, win, lo, hi: (win[j], 0)),
            pl.BlockSpec((BLK, D), lambda j, rel, win, lo, hi: (win[j], 0)),
        ],
        out_specs=pl.BlockSpec((BLK,), lambda j, rel, win, lo, hi: (win[j],)),
    )
    return pl.pallas_call(
        _tc_body,
        grid_spec=grid_spec,
        out_shape=jax.ShapeDtypeStruct((NPAD,), jnp.float32),
        compiler_params=pltpu.CompilerParams(
            dimension_semantics=("arbitrary",),
        ),
    )(item_rel, item_win, item_lo, item_hi, m3, h_pad, t_pad)


def _work_items(offs):
    off0 = offs[:-1]
    off1 = offs[1:]
    nonempty = off1 > off0
    w_start = off0 // BLK
    nwin = jnp.where(nonempty, (off1 - 1) // BLK - w_start + 1, 0)
    ends = jnp.cumsum(nwin)
    total = ends[-1]
    j = jnp.arange(WORK_N, dtype=jnp.int32)
    rid = jnp.searchsorted(ends, j, side="right").astype(jnp.int32)
    ridc = jnp.minimum(rid, R - 1)
    k = j - (ends[ridc] - nwin[ridc])
    valid = j < total
    item_rel = jnp.where(valid, ridc, 0)
    item_win = jnp.where(valid, w_start[ridc] + k, NPAD // BLK - 1)
    item_lo = jnp.where(valid, off0[ridc], 0)
    item_hi = jnp.where(valid, off1[ridc], 0)
    return item_rel, item_win, item_lo, item_hi


def kernel(head_ids, rel_ids, tail_ids, entity_table, relation_table):
    iota = jnp.arange(B, dtype=jnp.int32)
    skey = jnp.sort(rel_ids.astype(jnp.int32) * 32768 + iota)
    perm = skey & 32767
    srel = skey >> 15
    offs = jnp.searchsorted(srel, jnp.arange(R + 1, dtype=jnp.int32)).astype(jnp.int32)
    sorted_head = jnp.take(head_ids, perm).reshape(NWK, NCH, CH)
    sorted_tail = jnp.take(tail_ids, perm).reshape(NWK, NCH, CH)
    invperm = jnp.zeros((B,), jnp.int32).at[perm].set(iota)

    h_pad, t_pad = pl.kernel(
        _sc_gather_body,
        mesh=_sc_mesh(),
        out_type=[jax.ShapeDtypeStruct((NPAD, D), jnp.float32)] * 2,
        scratch_types=[
            pltpu.VMEM((NCH, CH), jnp.int32),
            pltpu.VMEM((BPW, D), jnp.float32),
            pltpu.SemaphoreType.DMA,
        ],
    )(entity_table, sorted_head, sorted_tail)

    m3 = relation_table.reshape(R, D, D)
    item_rel, item_win, item_lo, item_hi = _work_items(offs)
    scores_sorted = _grouped_scores(
        item_rel, item_win, item_lo, item_hi, m3, h_pad, t_pad
    )[:B]

    return jnp.take(scores_sorted, invperm)
